# baseline (speedup 1.0000x reference)
import jax
import jax.numpy as jnp
import numpy as np
from jax import lax
from jax.experimental import pallas as pl
from jax.experimental.pallas import tpu as pltpu
from jax.experimental.pallas import tpu_sc as plsc

_B = 16384
_L = 40
_V = 5
_E = 5
_FC = 200
_FCP = 256

_BSC = 4096
_BTC = _B - _BSC
_BB = 4096
_K = 4 * _L

_NW = 32
_PW = _BSC // _NW
_CH = 32
_NCHUNK = _FCP // 16


def _build_q_body(table_ref, wa_ref, wb_ref, ba_ref, bb_ref, q_ref, bias_ref):
    w = wa_ref[...] + wb_ref[...]
    ri = lax.broadcasted_iota(jnp.int32, (_FC, _FC), 0)
    ci = lax.broadcasted_iota(jnp.int32, (_FC, _FC), 1)
    blk = (ci // _E) == (ri // _V)
    sq = jnp.zeros((_FC, _FC), jnp.float32)
    for v in range(_V):
        rv = (ri % _V) == v
        for e in range(_E):
            m = blk & rv & ((ci % _E) == e)
            sq = jnp.where(m, table_ref[v, e], sq)
    q_ref[...] = lax.dot_general(
        sq, w, (((1,), (1,)), ((), ())), preferred_element_type=jnp.float32
    )
    bias_ref[...] = ba_ref[...] + bb_ref[...]


def _build_q(table, wap, wbp, bap, bbp):
    return pl.pallas_call(
        _build_q_body,
        in_specs=[
            pl.BlockSpec(memory_space=pltpu.SMEM),
            pl.BlockSpec((_FCP, _FC), lambda: (0, 0)),
            pl.BlockSpec((_FCP, _FC), lambda: (0, 0)),
            pl.BlockSpec((1, _FCP), lambda: (0, 0)),
            pl.BlockSpec((1, _FCP), lambda: (0, 0)),
        ],
        out_specs=[
            pl.BlockSpec((_FC, _FCP), lambda: (0, 0)),
            pl.BlockSpec((1, _FCP), lambda: (0, 0)),
        ],
        out_shape=[
            jax.ShapeDtypeStruct((_FC, _FCP), jnp.float32),
            jax.ShapeDtypeStruct((1, _FCP), jnp.float32),
        ],
    )(table, wap, wbp, bap, bbp)


def _sc_body(
    xf_hbm, q_hbm, bias_hbm, out_hbm,
    x_v, idx_v, rows0, rows1, bias_v, out_v, sem0, sem1,
):
    wid = lax.axis_index("s") * 2 + lax.axis_index("c")
    base = wid * _PW

    pltpu.sync_copy(xf_hbm.at[pl.ds(base * _L, _PW * _L)], x_v)
    pltpu.sync_copy(bias_hbm, bias_v)

    lane = lax.iota(jnp.int32, 16)

    @pl.loop(0, (_PW * _L) // 16)
    def _(j):
        k0 = j * 16
        lv = (k0 + lane) % _L
        idx_v[pl.ds(k0, 16)] = _V * lv + x_v[pl.ds(k0, 16)]

    def _fire(s, buf, sem):
        pltpu.async_copy(q_hbm.at[idx_v.at[pl.ds(s * _L, _L)]], buf, sem)

    def _wait(buf, sem):
        pltpu.make_async_copy(q_hbm.at[pl.ds(0, _L)], buf, sem).wait()

    def _acc_store(buf, t):
        for g in range(_FCP // 16):
            a = bias_v[pl.ds(16 * g, 16)]
            for l in range(_L):
                a = a + buf[l, pl.ds(16 * g, 16)]
            out_v[pl.ds(t * _FCP + 16 * g, 16)] = a

    @pl.loop(0, _PW // _CH)
    def _(blk):
        s_base = blk * _CH
        _fire(s_base, rows0, sem0)

        @pl.loop(0, _CH // 2)
        def _(i):
            s0 = s_base + 2 * i
            _fire(s0 + 1, rows1, sem1)
            _wait(rows0, sem0)
            _acc_store(rows0, 2 * i)

            @pl.when(i < _CH // 2 - 1)
            def _():
                _fire(s0 + 2, rows0, sem0)

            _wait(rows1, sem1)
            _acc_store(rows1, 2 * i + 1)

        pltpu.sync_copy(
            out_v, out_hbm.at[pl.ds((base + s_base) * _FCP, _CH * _FCP)]
        )


def _sc_bag(xf, qpad, biasv):
    return pl.kernel(
        _sc_body,
        out_type=jax.ShapeDtypeStruct((_BSC * _FCP,), jnp.float32),
        mesh=plsc.VectorSubcoreMesh(core_axis_name="c", subcore_axis_name="s"),
        scratch_types=[
            pltpu.VMEM((_PW * _L,), jnp.int32),
            pltpu.VMEM((_PW * _L,), jnp.int32),
            pltpu.VMEM((_L, _FCP), jnp.float32),
            pltpu.VMEM((_L, _FCP), jnp.float32),
            pltpu.VMEM((_FCP,), jnp.float32),
            pltpu.VMEM((_CH * _FCP,), jnp.float32),
            pltpu.SemaphoreType.DMA,
            pltpu.SemaphoreType.DMA,
        ],
    )(xf, qpad, biasv)


def _tc_body(inp_ref, table_ref, wa_ref, ba_ref, wb_ref, bb_ref, out_ref, g_ref, bias_ref):
    @pl.when(pl.program_id(0) == 0)
    def _():
        tt = [
            [table_ref[1, e] - table_ref[0, e] for e in range(_E)],
            [table_ref[2, e] - table_ref[0, e] for e in range(_E)],
            [table_ref[4, e] - table_ref[0, e] for e in range(_E)],
            [
                table_ref[0, e] - table_ref[1, e] - table_ref[2, e] + table_ref[3, e]
                for e in range(_E)
            ],
        ]
        w = wa_ref[...] + wb_ref[...]
        ri = lax.broadcasted_iota(jnp.int32, (_K, _FC), 0)
        ci = lax.broadcasted_iota(jnp.int32, (_K, _FC), 1)
        blk = (ci // _E) == (ri % _L)
        sg = jnp.zeros((_K, _FC), jnp.float32)
        for k in range(4):
            rk = (ri // _L) == k
            for e in range(_E):
                m = blk & rk & ((ci % _E) == e)
                sg = jnp.where(m, tt[k][e], sg)
        g_ref[...] = lax.dot_general(
            sg, w, (((1,), (1,)), ((), ())), preferred_element_type=jnp.float32
        ).astype(jnp.bfloat16)
        ci0 = lax.broadcasted_iota(jnp.int32, (1, _FC), 1)
        t0 = jnp.zeros((1, _FC), jnp.float32)
        for e in range(_E):
            t0 = jnp.where((ci0 % _E) == e, table_ref[0, e], t0)
        bias_ref[...] = (
            ba_ref[...]
            + bb_ref[...]
            + lax.dot_general(
                t0, w, (((1,), (1,)), ((), ())), preferred_element_type=jnp.float32
            )
        )

    x = inp_ref[...]
    b0 = x & 1
    b1 = (x >> 1) & 1
    p = b0 & b1
    b2 = (x >> 2) & 1
    f = jnp.concatenate([b0, b1, b2, p], axis=1).astype(jnp.bfloat16)
    out_ref[...] = (
        lax.dot_general(
            f, g_ref[...], (((1,), (0,)), ((), ())), preferred_element_type=jnp.float32
        )
        + bias_ref[...]
    )


def _tc_main(inp, table, W_a, b_a, W_b, b_b):
    grid = _BTC // _BB
    return pl.pallas_call(
        _tc_body,
        grid=(grid,),
        in_specs=[
            pl.BlockSpec((_BB, _L), lambda i: (i, 0)),
            pl.BlockSpec(memory_space=pltpu.SMEM),
            pl.BlockSpec((_FC, _FC), lambda i: (0, 0)),
            pl.BlockSpec((1, _FC), lambda i: (0, 0)),
            pl.BlockSpec((_FC, _FC), lambda i: (0, 0)),
            pl.BlockSpec((1, _FC), lambda i: (0, 0)),
        ],
        out_specs=pl.BlockSpec((_BB, _FC), lambda i: (i, 0)),
        out_shape=jax.ShapeDtypeStruct((_BTC, _FC), jnp.float32),
        scratch_shapes=[
            pltpu.VMEM((_K, _FC), jnp.bfloat16),
            pltpu.VMEM((1, _FC), jnp.float32),
        ],
    )(inp, table, W_a, b_a.reshape(1, _FC), W_b, b_b.reshape(1, _FC))


def kernel(input, table, W_a, b_a, W_b, b_b):
    x = input.astype(jnp.int32)
    zrows = jnp.zeros((_FCP - _FC, _FC), jnp.float32)
    wap = jnp.concatenate([W_a, zrows], axis=0)
    wbp = jnp.concatenate([W_b, zrows], axis=0)
    bap = jnp.concatenate([b_a, jnp.zeros((_FCP - _FC,), jnp.float32)]).reshape(1, _FCP)
    bbp = jnp.concatenate([b_b, jnp.zeros((_FCP - _FC,), jnp.float32)]).reshape(1, _FCP)
    qpad, biasp = _build_q(table, wap, wbp, bap, bbp)
    sc_flat = _sc_bag(x[_BTC:].reshape(-1), qpad, biasp.reshape(_FCP))
    tc_out = _tc_main(x[:_BTC], table, W_a, b_a, W_b, b_b)
    sc_out = sc_flat.reshape(_BSC, _FCP)[:, :_FC]
    return jnp.concatenate([tc_out, sc_out], axis=0)


# speedup vs baseline: 3.4622x; 3.4622x over previous
import jax
import jax.numpy as jnp
import numpy as np
from jax import lax
from jax.experimental import pallas as pl
from jax.experimental.pallas import tpu as pltpu
from jax.experimental.pallas import tpu_sc as plsc

_B = 16384
_L = 40
_V = 5
_E = 5
_FC = 200
_FCP = 256

_BSC = 512
_BTC = _B - _BSC
_BB = 4096
_K = 4 * _L

_NW = 32
_PW = _BSC // _NW
_CH = 16
_NCHUNK = _FCP // 16


def _build_q_body(table_ref, wa_ref, wb_ref, ba_ref, bb_ref, q_ref, bias_ref):
    w = wa_ref[...] + wb_ref[...]
    ri = lax.broadcasted_iota(jnp.int32, (_FC, _FC), 0)
    ci = lax.broadcasted_iota(jnp.int32, (_FC, _FC), 1)
    blk = (ci // _E) == (ri // _V)
    sq = jnp.zeros((_FC, _FC), jnp.float32)
    for v in range(_V):
        rv = (ri % _V) == v
        for e in range(_E):
            m = blk & rv & ((ci % _E) == e)
            sq = jnp.where(m, table_ref[v, e], sq)
    q_ref[...] = lax.dot_general(
        sq, w, (((1,), (1,)), ((), ())), preferred_element_type=jnp.float32
    )
    bias_ref[...] = ba_ref[...] + bb_ref[...]


def _build_q(table, wap, wbp, bap, bbp):
    return pl.pallas_call(
        _build_q_body,
        in_specs=[
            pl.BlockSpec(memory_space=pltpu.SMEM),
            pl.BlockSpec((_FCP, _FC), lambda: (0, 0)),
            pl.BlockSpec((_FCP, _FC), lambda: (0, 0)),
            pl.BlockSpec((1, _FCP), lambda: (0, 0)),
            pl.BlockSpec((1, _FCP), lambda: (0, 0)),
        ],
        out_specs=[
            pl.BlockSpec((_FC, _FCP), lambda: (0, 0)),
            pl.BlockSpec((1, _FCP), lambda: (0, 0)),
        ],
        out_shape=[
            jax.ShapeDtypeStruct((_FC, _FCP), jnp.float32),
            jax.ShapeDtypeStruct((1, _FCP), jnp.float32),
        ],
    )(table, wap, wbp, bap, bbp)


def _sc_body(
    xf_hbm, q_hbm, bias_hbm, out_hbm,
    x_v, idx_v, rows0, rows1, bias_v, out_v, sem0, sem1,
):
    wid = lax.axis_index("s") * 2 + lax.axis_index("c")
    base = wid * _PW

    pltpu.sync_copy(xf_hbm.at[pl.ds(base * _L, _PW * _L)], x_v)
    pltpu.sync_copy(bias_hbm, bias_v)

    lane = lax.iota(jnp.int32, 16)

    @pl.loop(0, (_PW * _L) // 16)
    def _(j):
        k0 = j * 16
        lv = (k0 + lane) % _L
        idx_v[pl.ds(k0, 16)] = _V * lv + x_v[pl.ds(k0, 16)]

    def _fire(s, buf, sem):
        pltpu.async_copy(q_hbm.at[idx_v.at[pl.ds(s * _L, _L)]], buf, sem)

    def _wait(buf, sem):
        pltpu.make_async_copy(q_hbm.at[pl.ds(0, _L)], buf, sem).wait()

    def _acc_store(buf, t):
        for g in range(_FCP // 16):
            a = bias_v[pl.ds(16 * g, 16)]
            for l in range(_L):
                a = a + buf[l, pl.ds(16 * g, 16)]
            out_v[pl.ds(t * _FCP + 16 * g, 16)] = a

    @pl.loop(0, _PW // _CH)
    def _(blk):
        s_base = blk * _CH
        _fire(s_base, rows0, sem0)

        @pl.loop(0, _CH // 2)
        def _(i):
            s0 = s_base + 2 * i
            _fire(s0 + 1, rows1, sem1)
            _wait(rows0, sem0)
            _acc_store(rows0, 2 * i)

            @pl.when(i < _CH // 2 - 1)
            def _():
                _fire(s0 + 2, rows0, sem0)

            _wait(rows1, sem1)
            _acc_store(rows1, 2 * i + 1)

        pltpu.sync_copy(
            out_v, out_hbm.at[pl.ds((base + s_base) * _FCP, _CH * _FCP)]
        )


def _sc_bag(xf, qpad, biasv):
    return pl.kernel(
        _sc_body,
        out_type=jax.ShapeDtypeStruct((_BSC * _FCP,), jnp.float32),
        mesh=plsc.VectorSubcoreMesh(core_axis_name="c", subcore_axis_name="s"),
        scratch_types=[
            pltpu.VMEM((_PW * _L,), jnp.int32),
            pltpu.VMEM((_PW * _L,), jnp.int32),
            pltpu.VMEM((_L, _FCP), jnp.float32),
            pltpu.VMEM((_L, _FCP), jnp.float32),
            pltpu.VMEM((_FCP,), jnp.float32),
            pltpu.VMEM((_CH * _FCP,), jnp.float32),
            pltpu.SemaphoreType.DMA,
            pltpu.SemaphoreType.DMA,
        ],
    )(xf, qpad, biasv)


def _tc_body(inp_ref, table_ref, wa_ref, ba_ref, wb_ref, bb_ref, out_ref, g_ref, bias_ref):
    @pl.when(pl.program_id(0) == 0)
    def _():
        tt = [
            [table_ref[1, e] - table_ref[0, e] for e in range(_E)],
            [table_ref[2, e] - table_ref[0, e] for e in range(_E)],
            [table_ref[4, e] - table_ref[0, e] for e in range(_E)],
            [
                table_ref[0, e] - table_ref[1, e] - table_ref[2, e] + table_ref[3, e]
                for e in range(_E)
            ],
        ]
        w = wa_ref[...] + wb_ref[...]
        ri = lax.broadcasted_iota(jnp.int32, (_K, _FC), 0)
        ci = lax.broadcasted_iota(jnp.int32, (_K, _FC), 1)
        blk = (ci // _E) == (ri % _L)
        sg = jnp.zeros((_K, _FC), jnp.float32)
        for k in range(4):
            rk = (ri // _L) == k
            for e in range(_E):
                m = blk & rk & ((ci % _E) == e)
                sg = jnp.where(m, tt[k][e], sg)
        g_ref[...] = lax.dot_general(
            sg, w, (((1,), (1,)), ((), ())), preferred_element_type=jnp.float32
        ).astype(jnp.bfloat16)
        ci0 = lax.broadcasted_iota(jnp.int32, (1, _FC), 1)
        t0 = jnp.zeros((1, _FC), jnp.float32)
        for e in range(_E):
            t0 = jnp.where((ci0 % _E) == e, table_ref[0, e], t0)
        bias_ref[...] = (
            ba_ref[...]
            + bb_ref[...]
            + lax.dot_general(
                t0, w, (((1,), (1,)), ((), ())), preferred_element_type=jnp.float32
            )
        )

    x = inp_ref[...]
    b0 = x & 1
    b1 = (x >> 1) & 1
    p = b0 & b1
    b2 = (x >> 2) & 1
    f = jnp.concatenate([b0, b1, b2, p], axis=1).astype(jnp.bfloat16)
    out_ref[...] = (
        lax.dot_general(
            f, g_ref[...], (((1,), (0,)), ((), ())), preferred_element_type=jnp.float32
        )
        + bias_ref[...]
    )


def _tc_main(inp, table, W_a, b_a, W_b, b_b):
    grid = pl.cdiv(_BTC, _BB)
    return pl.pallas_call(
        _tc_body,
        grid=(grid,),
        in_specs=[
            pl.BlockSpec((_BB, _L), lambda i: (i, 0)),
            pl.BlockSpec(memory_space=pltpu.SMEM),
            pl.BlockSpec((_FC, _FC), lambda i: (0, 0)),
            pl.BlockSpec((1, _FC), lambda i: (0, 0)),
            pl.BlockSpec((_FC, _FC), lambda i: (0, 0)),
            pl.BlockSpec((1, _FC), lambda i: (0, 0)),
        ],
        out_specs=pl.BlockSpec((_BB, _FC), lambda i: (i, 0)),
        out_shape=jax.ShapeDtypeStruct((_BTC, _FC), jnp.float32),
        scratch_shapes=[
            pltpu.VMEM((_K, _FC), jnp.bfloat16),
            pltpu.VMEM((1, _FC), jnp.float32),
        ],
    )(inp, table, W_a, b_a.reshape(1, _FC), W_b, b_b.reshape(1, _FC))


def kernel(input, table, W_a, b_a, W_b, b_b):
    x = input.astype(jnp.int32)
    zrows = jnp.zeros((_FCP - _FC, _FC), jnp.float32)
    wap = jnp.concatenate([W_a, zrows], axis=0)
    wbp = jnp.concatenate([W_b, zrows], axis=0)
    bap = jnp.concatenate([b_a, jnp.zeros((_FCP - _FC,), jnp.float32)]).reshape(1, _FCP)
    bbp = jnp.concatenate([b_b, jnp.zeros((_FCP - _FC,), jnp.float32)]).reshape(1, _FCP)
    qpad, biasp = _build_q(table, wap, wbp, bap, bbp)
    sc_flat = _sc_bag(x[_BTC:].reshape(-1), qpad, biasp.reshape(_FCP))
    tc_out = _tc_main(x[:_BTC], table, W_a, b_a, W_b, b_b)
    sc_out = sc_flat.reshape(_BSC, _FCP)[:, :_FC]
    return jnp.concatenate([tc_out, sc_out], axis=0)


# speedup vs baseline: 4.1128x; 1.1879x over previous
import jax
import jax.numpy as jnp
import numpy as np
from jax import lax
from jax.experimental import pallas as pl
from jax.experimental.pallas import tpu as pltpu
from jax.experimental.pallas import tpu_sc as plsc

_B = 16384
_L = 40
_V = 5
_E = 5
_FC = 200
_FCP = 256

_BSC = 256
_BTC = _B - _BSC
_BB = 4096
_K = 4 * _L

_NW = 32
_PW = _BSC // _NW
_CH = 8
_NCHUNK = _FCP // 16


def _build_q_body(table_ref, wa_ref, wb_ref, ba_ref, bb_ref, q_ref, bias_ref):
    w = wa_ref[...] + wb_ref[...]
    ri = lax.broadcasted_iota(jnp.int32, (_FC, _FC), 0)
    ci = lax.broadcasted_iota(jnp.int32, (_FC, _FC), 1)
    blk = (ci // _E) == (ri // _V)
    sq = jnp.zeros((_FC, _FC), jnp.float32)
    for v in range(_V):
        rv = (ri % _V) == v
        for e in range(_E):
            m = blk & rv & ((ci % _E) == e)
            sq = jnp.where(m, table_ref[v, e], sq)
    q_ref[...] = lax.dot_general(
        sq, w, (((1,), (1,)), ((), ())), preferred_element_type=jnp.float32
    )
    bias_ref[...] = ba_ref[...] + bb_ref[...]


def _build_q(table, wap, wbp, bap, bbp):
    return pl.pallas_call(
        _build_q_body,
        in_specs=[
            pl.BlockSpec(memory_space=pltpu.SMEM),
            pl.BlockSpec((_FCP, _FC), lambda: (0, 0)),
            pl.BlockSpec((_FCP, _FC), lambda: (0, 0)),
            pl.BlockSpec((1, _FCP), lambda: (0, 0)),
            pl.BlockSpec((1, _FCP), lambda: (0, 0)),
        ],
        out_specs=[
            pl.BlockSpec((_FC, _FCP), lambda: (0, 0)),
            pl.BlockSpec((1, _FCP), lambda: (0, 0)),
        ],
        out_shape=[
            jax.ShapeDtypeStruct((_FC, _FCP), jnp.float32),
            jax.ShapeDtypeStruct((1, _FCP), jnp.float32),
        ],
    )(table, wap, wbp, bap, bbp)


def _sc_body(
    xf_hbm, q_hbm, bias_hbm, out_hbm,
    x_v, idx_v, rows0, rows1, bias_v, out_v, sem0, sem1,
):
    wid = lax.axis_index("s") * 2 + lax.axis_index("c")
    base = wid * _PW

    pltpu.sync_copy(xf_hbm.at[pl.ds(base * _L, _PW * _L)], x_v)
    pltpu.sync_copy(bias_hbm, bias_v)

    lane = lax.iota(jnp.int32, 16)

    @pl.loop(0, (_PW * _L) // 16)
    def _(j):
        k0 = j * 16
        lv = (k0 + lane) % _L
        idx_v[pl.ds(k0, 16)] = _V * lv + x_v[pl.ds(k0, 16)]

    def _fire(s, buf, sem):
        pltpu.async_copy(q_hbm.at[idx_v.at[pl.ds(s * _L, _L)]], buf, sem)

    def _wait(buf, sem):
        pltpu.make_async_copy(q_hbm.at[pl.ds(0, _L)], buf, sem).wait()

    def _acc_store(buf, t):
        for g in range(_FCP // 16):
            a = bias_v[pl.ds(16 * g, 16)]
            for l in range(_L):
                a = a + buf[l, pl.ds(16 * g, 16)]
            out_v[pl.ds(t * _FCP + 16 * g, 16)] = a

    @pl.loop(0, _PW // _CH)
    def _(blk):
        s_base = blk * _CH
        _fire(s_base, rows0, sem0)

        @pl.loop(0, _CH // 2)
        def _(i):
            s0 = s_base + 2 * i
            _fire(s0 + 1, rows1, sem1)
            _wait(rows0, sem0)
            _acc_store(rows0, 2 * i)

            @pl.when(i < _CH // 2 - 1)
            def _():
                _fire(s0 + 2, rows0, sem0)

            _wait(rows1, sem1)
            _acc_store(rows1, 2 * i + 1)

        pltpu.sync_copy(
            out_v, out_hbm.at[pl.ds((base + s_base) * _FCP, _CH * _FCP)]
        )


def _sc_bag(xf, qpad, biasv):
    return pl.kernel(
        _sc_body,
        out_type=jax.ShapeDtypeStruct((_BSC * _FCP,), jnp.float32),
        mesh=plsc.VectorSubcoreMesh(core_axis_name="c", subcore_axis_name="s"),
        scratch_types=[
            pltpu.VMEM((_PW * _L,), jnp.int32),
            pltpu.VMEM((_PW * _L,), jnp.int32),
            pltpu.VMEM((_L, _FCP), jnp.float32),
            pltpu.VMEM((_L, _FCP), jnp.float32),
            pltpu.VMEM((_FCP,), jnp.float32),
            pltpu.VMEM((_CH * _FCP,), jnp.float32),
            pltpu.SemaphoreType.DMA,
            pltpu.SemaphoreType.DMA,
        ],
    )(xf, qpad, biasv)


def _tc_body(inp_ref, table_ref, wa_ref, ba_ref, wb_ref, bb_ref, out_ref, g_ref, bias_ref):
    @pl.when(pl.program_id(0) == 0)
    def _():
        tt = [
            [table_ref[1, e] - table_ref[0, e] for e in range(_E)],
            [table_ref[2, e] - table_ref[0, e] for e in range(_E)],
            [table_ref[4, e] - table_ref[0, e] for e in range(_E)],
            [
                table_ref[0, e] - table_ref[1, e] - table_ref[2, e] + table_ref[3, e]
                for e in range(_E)
            ],
        ]
        w = wa_ref[...] + wb_ref[...]
        ri = lax.broadcasted_iota(jnp.int32, (_K, _FC), 0)
        ci = lax.broadcasted_iota(jnp.int32, (_K, _FC), 1)
        blk = (ci // _E) == (ri % _L)
        sg = jnp.zeros((_K, _FC), jnp.float32)
        for k in range(4):
            rk = (ri // _L) == k
            for e in range(_E):
                m = blk & rk & ((ci % _E) == e)
                sg = jnp.where(m, tt[k][e], sg)
        g_ref[...] = lax.dot_general(
            sg, w, (((1,), (1,)), ((), ())), preferred_element_type=jnp.float32
        ).astype(jnp.bfloat16)
        ci0 = lax.broadcasted_iota(jnp.int32, (1, _FC), 1)
        t0 = jnp.zeros((1, _FC), jnp.float32)
        for e in range(_E):
            t0 = jnp.where((ci0 % _E) == e, table_ref[0, e], t0)
        bias_ref[...] = (
            ba_ref[...]
            + bb_ref[...]
            + lax.dot_general(
                t0, w, (((1,), (1,)), ((), ())), preferred_element_type=jnp.float32
            )
        )

    x = inp_ref[...]
    b0 = x & 1
    b1 = (x >> 1) & 1
    p = b0 & b1
    b2 = (x >> 2) & 1
    f = jnp.concatenate([b0, b1, b2, p], axis=1).astype(jnp.bfloat16)
    out_ref[...] = (
        lax.dot_general(
            f, g_ref[...], (((1,), (0,)), ((), ())), preferred_element_type=jnp.float32
        )
        + bias_ref[...]
    )


def _tc_main(inp, table, W_a, b_a, W_b, b_b):
    grid = pl.cdiv(_BTC, _BB)
    return pl.pallas_call(
        _tc_body,
        grid=(grid,),
        in_specs=[
            pl.BlockSpec((_BB, _L), lambda i: (i, 0)),
            pl.BlockSpec(memory_space=pltpu.SMEM),
            pl.BlockSpec((_FC, _FC), lambda i: (0, 0)),
            pl.BlockSpec((1, _FC), lambda i: (0, 0)),
            pl.BlockSpec((_FC, _FC), lambda i: (0, 0)),
            pl.BlockSpec((1, _FC), lambda i: (0, 0)),
        ],
        out_specs=pl.BlockSpec((_BB, _FC), lambda i: (i, 0)),
        out_shape=jax.ShapeDtypeStruct((_BTC, _FC), jnp.float32),
        scratch_shapes=[
            pltpu.VMEM((_K, _FC), jnp.bfloat16),
            pltpu.VMEM((1, _FC), jnp.float32),
        ],
    )(inp, table, W_a, b_a.reshape(1, _FC), W_b, b_b.reshape(1, _FC))


def kernel(input, table, W_a, b_a, W_b, b_b):
    x = input.astype(jnp.int32)
    zrows = jnp.zeros((_FCP - _FC, _FC), jnp.float32)
    wap = jnp.concatenate([W_a, zrows], axis=0)
    wbp = jnp.concatenate([W_b, zrows], axis=0)
    bap = jnp.concatenate([b_a, jnp.zeros((_FCP - _FC,), jnp.float32)]).reshape(1, _FCP)
    bbp = jnp.concatenate([b_b, jnp.zeros((_FCP - _FC,), jnp.float32)]).reshape(1, _FCP)
    qpad, biasp = _build_q(table, wap, wbp, bap, bbp)
    sc_flat = _sc_bag(x[_BTC:].reshape(-1), qpad, biasp.reshape(_FCP))
    tc_out = _tc_main(x[:_BTC], table, W_a, b_a, W_b, b_b)
    sc_out = sc_flat.reshape(_BSC, _FCP)[:, :_FC]
    return jnp.concatenate([tc_out, sc_out], axis=0)


# speedup vs baseline: 4.6684x; 1.1351x over previous
import jax
import jax.numpy as jnp
import numpy as np
from jax import lax
from jax.experimental import pallas as pl
from jax.experimental.pallas import tpu as pltpu
from jax.experimental.pallas import tpu_sc as plsc

_B = 16384
_L = 40
_V = 5
_E = 5
_FC = 200
_FCP = 256

_BSC = 128
_BTC = _B - _BSC
_BB = 4096
_K = 4 * _L

_NW = 32
_PW = _BSC // _NW
_CH = 4
_NCHUNK = _FCP // 16


def _build_q_body(table_ref, wa_ref, wb_ref, ba_ref, bb_ref, q_ref, bias_ref):
    w = wa_ref[...] + wb_ref[...]
    ri = lax.broadcasted_iota(jnp.int32, (_FC, _FC), 0)
    ci = lax.broadcasted_iota(jnp.int32, (_FC, _FC), 1)
    blk = (ci // _E) == (ri // _V)
    sq = jnp.zeros((_FC, _FC), jnp.float32)
    for v in range(_V):
        rv = (ri % _V) == v
        for e in range(_E):
            m = blk & rv & ((ci % _E) == e)
            sq = jnp.where(m, table_ref[v, e], sq)
    q_ref[...] = lax.dot_general(
        sq, w, (((1,), (1,)), ((), ())), preferred_element_type=jnp.float32
    )
    bias_ref[...] = ba_ref[...] + bb_ref[...]


def _build_q(table, wap, wbp, bap, bbp):
    return pl.pallas_call(
        _build_q_body,
        in_specs=[
            pl.BlockSpec(memory_space=pltpu.SMEM),
            pl.BlockSpec((_FCP, _FC), lambda: (0, 0)),
            pl.BlockSpec((_FCP, _FC), lambda: (0, 0)),
            pl.BlockSpec((1, _FCP), lambda: (0, 0)),
            pl.BlockSpec((1, _FCP), lambda: (0, 0)),
        ],
        out_specs=[
            pl.BlockSpec((_FC, _FCP), lambda: (0, 0)),
            pl.BlockSpec((1, _FCP), lambda: (0, 0)),
        ],
        out_shape=[
            jax.ShapeDtypeStruct((_FC, _FCP), jnp.float32),
            jax.ShapeDtypeStruct((1, _FCP), jnp.float32),
        ],
    )(table, wap, wbp, bap, bbp)


def _sc_body(
    xf_hbm, q_hbm, bias_hbm, out_hbm,
    x_v, idx_v, rows0, rows1, bias_v, out_v, sem0, sem1,
):
    wid = lax.axis_index("s") * 2 + lax.axis_index("c")
    base = wid * _PW

    pltpu.sync_copy(xf_hbm.at[pl.ds(base * _L, _PW * _L)], x_v)
    pltpu.sync_copy(bias_hbm, bias_v)

    lane = lax.iota(jnp.int32, 16)

    @pl.loop(0, (_PW * _L) // 16)
    def _(j):
        k0 = j * 16
        lv = (k0 + lane) % _L
        idx_v[pl.ds(k0, 16)] = _V * lv + x_v[pl.ds(k0, 16)]

    def _fire(s, buf, sem):
        pltpu.async_copy(q_hbm.at[idx_v.at[pl.ds(s * _L, _L)]], buf, sem)

    def _wait(buf, sem):
        pltpu.make_async_copy(q_hbm.at[pl.ds(0, _L)], buf, sem).wait()

    def _acc_store(buf, t):
        for g in range(_FCP // 16):
            a = bias_v[pl.ds(16 * g, 16)]
            for l in range(_L):
                a = a + buf[l, pl.ds(16 * g, 16)]
            out_v[pl.ds(t * _FCP + 16 * g, 16)] = a

    @pl.loop(0, _PW // _CH)
    def _(blk):
        s_base = blk * _CH
        _fire(s_base, rows0, sem0)

        @pl.loop(0, _CH // 2)
        def _(i):
            s0 = s_base + 2 * i
            _fire(s0 + 1, rows1, sem1)
            _wait(rows0, sem0)
            _acc_store(rows0, 2 * i)

            @pl.when(i < _CH // 2 - 1)
            def _():
                _fire(s0 + 2, rows0, sem0)

            _wait(rows1, sem1)
            _acc_store(rows1, 2 * i + 1)

        pltpu.sync_copy(
            out_v, out_hbm.at[pl.ds((base + s_base) * _FCP, _CH * _FCP)]
        )


def _sc_bag(xf, qpad, biasv):
    return pl.kernel(
        _sc_body,
        out_type=jax.ShapeDtypeStruct((_BSC * _FCP,), jnp.float32),
        mesh=plsc.VectorSubcoreMesh(core_axis_name="c", subcore_axis_name="s"),
        scratch_types=[
            pltpu.VMEM((_PW * _L,), jnp.int32),
            pltpu.VMEM((_PW * _L,), jnp.int32),
            pltpu.VMEM((_L, _FCP), jnp.float32),
            pltpu.VMEM((_L, _FCP), jnp.float32),
            pltpu.VMEM((_FCP,), jnp.float32),
            pltpu.VMEM((_CH * _FCP,), jnp.float32),
            pltpu.SemaphoreType.DMA,
            pltpu.SemaphoreType.DMA,
        ],
    )(xf, qpad, biasv)


def _tc_body(inp_ref, table_ref, wa_ref, ba_ref, wb_ref, bb_ref, out_ref, g_ref, bias_ref):
    @pl.when(pl.program_id(0) == 0)
    def _():
        tt = [
            [table_ref[1, e] - table_ref[0, e] for e in range(_E)],
            [table_ref[2, e] - table_ref[0, e] for e in range(_E)],
            [table_ref[4, e] - table_ref[0, e] for e in range(_E)],
            [
                table_ref[0, e] - table_ref[1, e] - table_ref[2, e] + table_ref[3, e]
                for e in range(_E)
            ],
        ]
        w = wa_ref[...] + wb_ref[...]
        ri = lax.broadcasted_iota(jnp.int32, (_K, _FC), 0)
        ci = lax.broadcasted_iota(jnp.int32, (_K, _FC), 1)
        blk = (ci // _E) == (ri % _L)
        sg = jnp.zeros((_K, _FC), jnp.float32)
        for k in range(4):
            rk = (ri // _L) == k
            for e in range(_E):
                m = blk & rk & ((ci % _E) == e)
                sg = jnp.where(m, tt[k][e], sg)
        g_ref[...] = lax.dot_general(
            sg, w, (((1,), (1,)), ((), ())), preferred_element_type=jnp.float32
        ).astype(jnp.bfloat16)
        ci0 = lax.broadcasted_iota(jnp.int32, (1, _FC), 1)
        t0 = jnp.zeros((1, _FC), jnp.float32)
        for e in range(_E):
            t0 = jnp.where((ci0 % _E) == e, table_ref[0, e], t0)
        bias_ref[...] = (
            ba_ref[...]
            + bb_ref[...]
            + lax.dot_general(
                t0, w, (((1,), (1,)), ((), ())), preferred_element_type=jnp.float32
            )
        )

    x = inp_ref[...]
    b0 = x & 1
    b1 = (x >> 1) & 1
    p = b0 & b1
    b2 = (x >> 2) & 1
    f = jnp.concatenate([b0, b1, b2, p], axis=1).astype(jnp.bfloat16)
    out_ref[...] = (
        lax.dot_general(
            f, g_ref[...], (((1,), (0,)), ((), ())), preferred_element_type=jnp.float32
        )
        + bias_ref[...]
    )


def _tc_main(inp, table, W_a, b_a, W_b, b_b):
    grid = pl.cdiv(_BTC, _BB)
    return pl.pallas_call(
        _tc_body,
        grid=(grid,),
        in_specs=[
            pl.BlockSpec((_BB, _L), lambda i: (i, 0)),
            pl.BlockSpec(memory_space=pltpu.SMEM),
            pl.BlockSpec((_FC, _FC), lambda i: (0, 0)),
            pl.BlockSpec((1, _FC), lambda i: (0, 0)),
            pl.BlockSpec((_FC, _FC), lambda i: (0, 0)),
            pl.BlockSpec((1, _FC), lambda i: (0, 0)),
        ],
        out_specs=pl.BlockSpec((_BB, _FC), lambda i: (i, 0)),
        out_shape=jax.ShapeDtypeStruct((_BTC, _FC), jnp.float32),
        scratch_shapes=[
            pltpu.VMEM((_K, _FC), jnp.bfloat16),
            pltpu.VMEM((1, _FC), jnp.float32),
        ],
    )(inp, table, W_a, b_a.reshape(1, _FC), W_b, b_b.reshape(1, _FC))


def kernel(input, table, W_a, b_a, W_b, b_b):
    x = input.astype(jnp.int32)
    zrows = jnp.zeros((_FCP - _FC, _FC), jnp.float32)
    wap = jnp.concatenate([W_a, zrows], axis=0)
    wbp = jnp.concatenate([W_b, zrows], axis=0)
    bap = jnp.concatenate([b_a, jnp.zeros((_FCP - _FC,), jnp.float32)]).reshape(1, _FCP)
    bbp = jnp.concatenate([b_b, jnp.zeros((_FCP - _FC,), jnp.float32)]).reshape(1, _FCP)
    qpad, biasp = _build_q(table, wap, wbp, bap, bbp)
    sc_flat = _sc_bag(x[_BTC:].reshape(-1), qpad, biasp.reshape(_FCP))
    tc_out = _tc_main(x[:_BTC], table, W_a, b_a, W_b, b_b)
    sc_out = sc_flat.reshape(_BSC, _FCP)[:, :_FC]
    return jnp.concatenate([tc_out, sc_out], axis=0)
